# baseline (speedup 1.0000x reference)
import functools

import jax
import jax.numpy as jnp
import numpy as np
from jax import lax
from jax.experimental import pallas as pl
from jax.experimental.pallas import tpu as pltpu
from jax.experimental.pallas import tpu_sc as plsc

N = 10000
K = 32
NK = N * K
D_IN = 128
D_OUT = 128
H = 64
EPS = 1e-5

_NC = 2
_NS = 16
_NW = _NC * _NS
_CH = 128
_GRP = 4
_CPW = 80
_NKP = _NW * _CPW * _CH

_T = 80
_R = _T * K
_GRID = N // _T
_TE = 2000
_TF = 400

_F32 = jnp.float32
_BF16 = jnp.bfloat16
_HI = lax.Precision.HIGHEST


def _gather_rows(table, idx3, d):
  mesh = plsc.VectorSubcoreMesh(core_axis_name="c", subcore_axis_name="s")

  @functools.partial(
      pl.kernel,
      out_type=jax.ShapeDtypeStruct((_NKP, d), _F32),
      mesh=mesh,
      scratch_types=[
          pltpu.VMEM((_CPW, _CH), jnp.int32),
          pltpu.VMEM((_GRP * _CH, d), _F32),
          pltpu.SemaphoreType.DMA,
      ],
      compiler_params=pltpu.CompilerParams(use_tc_tiling_on_sc=False),
  )
  def k(table_hbm, idx_hbm, out_hbm, idx_v, rows_v, sem):
    cid = lax.axis_index("c")
    sid = lax.axis_index("s")
    wid = sid * _NC + cid
    base = wid * _CPW * _CH
    pltpu.sync_copy(idx_hbm.at[wid], idx_v)

    def body(g, carry):
      cps = []
      for b in range(_GRP):
        cps.append(pltpu.async_copy(
            table_hbm.at[idx_v.at[g * _GRP + b]],
            rows_v.at[pl.ds(b * _CH, _CH)], sem))
      for cp in cps:
        cp.wait()
      pltpu.sync_copy(rows_v,
                      out_hbm.at[pl.ds(base + g * (_GRP * _CH), _GRP * _CH)])
      return carry

    lax.fori_loop(0, _CPW // _GRP, body, 0)

  return k(table, idx3)


_A_NP = np.zeros((16, 16), np.float32)
_B_NP = np.zeros((16, 16), np.float32)
for _i in range(3):
  _A_NP[_i, 1 + _i] = 1.0
  _A_NP[_i, 4 + _i] = 1.0
  _B_NP[_i, 1 + _i] = -1.0
  _B_NP[_i, 7 + _i] = 1.0


def _dotT(a, b):
  return lax.dot_general(a, b, (((1,), (1,)), ((), ())),
                         preferred_element_type=_F32, precision=_HI)


def _rel_dis(xt, nb):
  xtr = jnp.broadcast_to(xt[:, None, :], (_T, K, 16)).reshape(_R, 16)
  rel = xtr - nb
  dis = jnp.sqrt(jnp.sum(rel * rel, axis=1, keepdims=True))
  return rel, dis


def _kernel_ab(x_ref, xyz_ref, nxyz_ref, w1_ref, am_ref, bm_ref,
               y0_ref, s0_ref, sx_ref, m1_ref, s1_ref):
  i = pl.program_id(0)
  x = x_ref[...]
  y0_ref[...] = _dotT(x, w1_ref[...])

  @pl.when(i == 0)
  def _():
    s0_ref[...] = jnp.zeros_like(s0_ref)
    sx_ref[...] = jnp.zeros_like(sx_ref)
    m1_ref[...] = jnp.zeros_like(m1_ref)
    s1_ref[...] = jnp.zeros_like(s1_ref)

  s0_ref[...] += lax.dot_general(x, x, (((0,), (0,)), ((), ())),
                                 preferred_element_type=_F32, precision=_HI)
  sx_ref[...] += jnp.sum(x, axis=0, keepdims=True)

  nb = nxyz_ref[...]
  xt = xyz_ref[...]
  xtr = jnp.broadcast_to(xt[:, None, :], (_T, K, 16)).reshape(_R, 16)
  rel = xtr - nb
  dis = jnp.sqrt(jnp.sum(rel * rel, axis=1, keepdims=True))
  li = lax.broadcasted_iota(jnp.int32, (_R, 16), 1)
  f10 = (lax.dot_general(xtr, am_ref[...], (((1,), (0,)), ((), ())),
                         preferred_element_type=_F32, precision=_HI)
         + lax.dot_general(nb, bm_ref[...], (((1,), (0,)), ((), ())),
                           preferred_element_type=_F32, precision=_HI)
         + jnp.where(li == 0, dis, 0.0))
  m1_ref[...] += lax.dot_general(f10, f10, (((0,), (0,)), ((), ())),
                                 preferred_element_type=_F32, precision=_HI)
  s1_ref[...] += jnp.sum(f10, axis=0, keepdims=True)


def _fxyz(xyz_ref, nxyz_ref, wrel_ref, wtile_ref, wnb_ref, w0_ref,
          a1_ref, c1_ref):
  nb = nxyz_ref[...]
  xt = xyz_ref[...]
  rel, dis = _rel_dis(xt, nb)
  px = lax.dot_general(xt, wtile_ref[...], (((1,), (0,)), ((), ())),
                       preferred_element_type=_F32, precision=_HI)
  pxb = jnp.broadcast_to(px[:, None, :], (_T, K, H)).reshape(_R, H)
  prel = lax.dot_general(rel, wrel_ref[...], (((1,), (0,)), ((), ())),
                         preferred_element_type=_F32, precision=_HI)
  pn = lax.dot_general(nb, wnb_ref[...], (((1,), (0,)), ((), ())),
                       preferred_element_type=_F32, precision=_HI)
  y1 = pxb + prel + pn + dis * w0_ref[...]
  return jax.nn.relu(y1 * a1_ref[...] + c1_ref[...])


def _att_pool_tile(fn, fx, wfca_ref, wfcb_ref, wml_ref, wmr_ref):
  att = _dotT(fn, wfca_ref[...]) + _dotT(fx, wfcb_ref[...])
  a3 = att.reshape(_T, K, D_OUT)
  m = jnp.max(a3, axis=1, keepdims=True)
  e = jnp.exp(a3 - m)
  sc = e / jnp.sum(e, axis=1, keepdims=True)
  fn3 = fn.reshape(_T, K, H)
  fx3 = fx.reshape(_T, K, H)
  agg_l = jnp.sum(fn3 * sc[:, :, :H], axis=1)
  agg_r = jnp.sum(fx3 * sc[:, :, H:], axis=1)
  return _dotT(agg_l, wml_ref[...]) + _dotT(agg_r, wmr_ref[...])


def _kernel_att1(xyz_ref, nxyz_ref, nb1_ref, wrel_ref, wtile_ref, wnb_ref,
                 w0_ref, a1_ref, c1_ref, a0_ref, c0_ref,
                 wfca_ref, wfcb_ref, wml_ref, wmr_ref,
                 z1_ref, sz_ref, qz_ref, m2_ref, s2_ref):
  i = pl.program_id(0)
  fx = _fxyz(xyz_ref, nxyz_ref, wrel_ref, wtile_ref, wnb_ref, w0_ref,
             a1_ref, c1_ref)
  fn = jax.nn.relu(nb1_ref[...] * a0_ref[...] + c0_ref[...])
  z1 = _att_pool_tile(fn, fx, wfca_ref, wfcb_ref, wml_ref, wmr_ref)
  z1_ref[...] = z1

  @pl.when(i == 0)
  def _():
    sz_ref[...] = jnp.zeros_like(sz_ref)
    qz_ref[...] = jnp.zeros_like(qz_ref)
    m2_ref[...] = jnp.zeros_like(m2_ref)
    s2_ref[...] = jnp.zeros_like(s2_ref)

  sz_ref[...] += jnp.sum(z1, axis=0, keepdims=True)
  qz_ref[...] += jnp.sum(z1 * z1, axis=0, keepdims=True)
  m2_ref[...] += lax.dot_general(fx, fx, (((0,), (0,)), ((), ())),
                                 preferred_element_type=_F32, precision=_HI)
  s2_ref[...] += jnp.sum(fx, axis=0, keepdims=True)


def _kernel_att2(xyz_ref, nxyz_ref, nb2_ref, wrel_ref, wtile_ref, wnb_ref,
                 w0_ref, a1_ref, c1_ref, w2_ref, a2_ref, c2_ref,
                 az_ref, cz_ref, wfca_ref, wfcb_ref, wml_ref, wmr_ref,
                 z2_ref, sz_ref, qz_ref):
  i = pl.program_id(0)
  fx1 = _fxyz(xyz_ref, nxyz_ref, wrel_ref, wtile_ref, wnb_ref, w0_ref,
              a1_ref, c1_ref)
  fx2 = jax.nn.relu(_dotT(fx1, w2_ref[...]) * a2_ref[...] + c2_ref[...])
  fn2 = jax.nn.relu(nb2_ref[...] * az_ref[...] + cz_ref[...])
  z2 = _att_pool_tile(fn2, fx2, wfca_ref, wfcb_ref, wml_ref, wmr_ref)
  z2_ref[...] = z2

  @pl.when(i == 0)
  def _():
    sz_ref[...] = jnp.zeros_like(sz_ref)
    qz_ref[...] = jnp.zeros_like(qz_ref)

  sz_ref[...] += jnp.sum(z2, axis=0, keepdims=True)
  qz_ref[...] += jnp.sum(z2 * z2, axis=0, keepdims=True)


def _kernel_stats3(z2_ref, az2_ref, cz2_ref, m3_ref, s3_ref):
  i = pl.program_id(0)
  fa = jax.nn.relu(z2_ref[...] * az2_ref[...] + cz2_ref[...])

  @pl.when(i == 0)
  def _():
    m3_ref[...] = jnp.zeros_like(m3_ref)
    s3_ref[...] = jnp.zeros_like(s3_ref)

  m3_ref[...] += lax.dot_general(fa, fa, (((0,), (0,)), ((), ())),
                                 preferred_element_type=_F32, precision=_HI)
  s3_ref[...] += jnp.sum(fa, axis=0, keepdims=True)


def _kernel_final(z2_ref, x_ref, az2_ref, cz2_ref, w3_ref, a3_ref, b3_ref,
                  wsc_ref, asc_ref, bsc_ref, out_ref):
  fa = jax.nn.relu(z2_ref[...] * az2_ref[...] + cz2_ref[...])
  y3 = _dotT(fa, w3_ref[...]) * a3_ref[...] + b3_ref[...]
  ysc = _dotT(x_ref[...], wsc_ref[...]) * asc_ref[...] + bsc_ref[...]
  o = y3 + ysc
  out_ref[...] = jnp.where(o >= 0, o, 0.2 * o)


def _full(shape):
  return pl.BlockSpec(shape, lambda i: (0,) * len(shape))


def _fold_moment(w, g, b, s2m, s1m, n):
  mu = s1m / n
  mean_y = jnp.matmul(w, mu, precision=_HI)
  ey2 = jnp.sum(jnp.matmul(w, s2m / n, precision=_HI) * w, axis=1)
  var = jnp.maximum(ey2 - mean_y * mean_y, 0.0)
  a = g * lax.rsqrt(var + EPS)
  return a, b - mean_y * a


def _fold_direct(g, b, ssum, sq, n):
  mean = ssum / n
  var = jnp.maximum(sq / n - mean * mean, 0.0)
  a = g * lax.rsqrt(var + EPS)
  return a, b - mean * a


def kernel(feature, xyz, neigh_idx, W_mlp1, g_mlp1, b_mlp1, W_bb1, g_bb1,
           b_bb1, W_att1_fc, W_att1_mlp, g_att1, b_att1, W_bb2, g_bb2, b_bb2,
           W_att2_fc, W_att2_mlp, g_att2, b_att2, W_mlp2, g_mlp2, b_mlp2,
           W_sc, g_sc, b_sc):
  x = feature[0, :, :, 0].T
  xyz16 = jnp.pad(xyz[0], ((0, 0), (0, 13)))
  idxf = neigh_idx[0].reshape(-1).astype(jnp.int32)
  idx3 = jnp.pad(idxf, (0, _NKP - NK)).reshape(_NW, _CPW, _CH)

  nxyz = _gather_rows(xyz16, idx3, 16)

  grid_nk = lambda i: (i, 0)
  y0, s0, sx, m1, s1 = pl.pallas_call(
      _kernel_ab,
      grid=(_GRID,),
      in_specs=[
          pl.BlockSpec((_T, D_IN), grid_nk),
          pl.BlockSpec((_T, 16), grid_nk),
          pl.BlockSpec((_R, 16), grid_nk),
          _full((H, D_IN)),
          _full((16, 16)), _full((16, 16)),
      ],
      out_specs=[
          pl.BlockSpec((_T, H), grid_nk),
          _full((D_IN, D_IN)),
          _full((1, D_IN)),
          _full((16, 16)),
          _full((1, 16)),
      ],
      out_shape=[
          jax.ShapeDtypeStruct((N, H), _F32),
          jax.ShapeDtypeStruct((D_IN, D_IN), _F32),
          jax.ShapeDtypeStruct((1, D_IN), _F32),
          jax.ShapeDtypeStruct((16, 16), _F32),
          jax.ShapeDtypeStruct((1, 16), _F32),
      ],
  )(x, xyz16, nxyz, W_mlp1,
    jnp.asarray(_A_NP), jnp.asarray(_B_NP))

  a0, c0 = _fold_moment(W_mlp1, g_mlp1, b_mlp1, s0, sx[0], N)
  w1p = jnp.pad(W_bb1, ((0, 0), (0, 6)))
  a1, c1 = _fold_moment(w1p, g_bb1, b_bb1, m1, s1[0], NK)
  w1b = W_bb1
  zpad = jnp.zeros((13, H), _F32)
  wrel = jnp.concatenate([w1b[:, 1:4].T, zpad], axis=0)
  wtile = jnp.concatenate([w1b[:, 4:7].T, zpad], axis=0)
  wnb = jnp.concatenate([w1b[:, 7:10].T, zpad], axis=0)
  w0row = w1b[:, 0][None, :]

  nb1 = _gather_rows(y0, idx3, H)

  z1, sz1, qz1, m2, s2 = pl.pallas_call(
      _kernel_att1,
      grid=(_GRID,),
      in_specs=[
          pl.BlockSpec((_T, 16), grid_nk),
          pl.BlockSpec((_R, 16), grid_nk),
          pl.BlockSpec((_R, H), grid_nk),
          _full((16, H)), _full((16, H)), _full((16, H)), _full((1, H)),
          _full((1, H)), _full((1, H)),
          _full((1, H)), _full((1, H)),
          _full((D_OUT, H)), _full((D_OUT, H)),
          _full((H, H)), _full((H, H)),
      ],
      out_specs=[
          pl.BlockSpec((_T, H), grid_nk),
          _full((1, H)), _full((1, H)), _full((H, H)), _full((1, H)),
      ],
      out_shape=[
          jax.ShapeDtypeStruct((N, H), _F32),
          jax.ShapeDtypeStruct((1, H), _F32),
          jax.ShapeDtypeStruct((1, H), _F32),
          jax.ShapeDtypeStruct((H, H), _F32),
          jax.ShapeDtypeStruct((1, H), _F32),
      ],
  )(xyz16, nxyz, nb1, wrel, wtile, wnb, w0row,
    a1[None, :], c1[None, :], a0[None, :], c0[None, :],
    W_att1_fc[:, :H], W_att1_fc[:, H:],
    W_att1_mlp[:, :H], W_att1_mlp[:, H:])

  az1, cz1 = _fold_direct(g_att1, b_att1, sz1[0], qz1[0], N)
  a2, c2 = _fold_moment(W_bb2, g_bb2, b_bb2, m2, s2[0], NK)

  nb2 = _gather_rows(z1, idx3, H)

  z2, sz2, qz2 = pl.pallas_call(
      _kernel_att2,
      grid=(_GRID,),
      in_specs=[
          pl.BlockSpec((_T, 16), grid_nk),
          pl.BlockSpec((_R, 16), grid_nk),
          pl.BlockSpec((_R, H), grid_nk),
          _full((16, H)), _full((16, H)), _full((16, H)), _full((1, H)),
          _full((1, H)), _full((1, H)),
          _full((H, H)), _full((1, H)), _full((1, H)),
          _full((1, H)), _full((1, H)),
          _full((D_OUT, H)), _full((D_OUT, H)),
          _full((D_OUT, H)), _full((D_OUT, H)),
      ],
      out_specs=[
          pl.BlockSpec((_T, D_OUT), grid_nk),
          _full((1, D_OUT)), _full((1, D_OUT)),
      ],
      out_shape=[
          jax.ShapeDtypeStruct((N, D_OUT), _F32),
          jax.ShapeDtypeStruct((1, D_OUT), _F32),
          jax.ShapeDtypeStruct((1, D_OUT), _F32),
      ],
  )(xyz16, nxyz, nb2, wrel, wtile, wnb, w0row,
    a1[None, :], c1[None, :],
    W_bb2, a2[None, :], c2[None, :],
    az1[None, :], cz1[None, :],
    W_att2_fc[:, :H], W_att2_fc[:, H:],
    W_att2_mlp[:, :H], W_att2_mlp[:, H:])

  az2, cz2 = _fold_direct(g_att2, b_att2, sz2[0], qz2[0], N)

  m3, s3 = pl.pallas_call(
      _kernel_stats3,
      grid=(N // _TE,),
      in_specs=[
          pl.BlockSpec((_TE, D_OUT), grid_nk),
          _full((1, D_OUT)), _full((1, D_OUT)),
      ],
      out_specs=[_full((D_OUT, D_OUT)), _full((1, D_OUT))],
      out_shape=[
          jax.ShapeDtypeStruct((D_OUT, D_OUT), _F32),
          jax.ShapeDtypeStruct((1, D_OUT), _F32),
      ],
  )(z2, az2[None, :], cz2[None, :])

  a3, c3 = _fold_moment(W_mlp2, g_mlp2, b_mlp2, m3, s3[0], N)
  asc, csc = _fold_moment(W_sc, g_sc, b_sc, s0, sx[0], N)

  out = pl.pallas_call(
      _kernel_final,
      grid=(N // _TF,),
      in_specs=[
          pl.BlockSpec((_TF, D_OUT), grid_nk),
          pl.BlockSpec((_TF, D_IN), grid_nk),
          _full((1, D_OUT)), _full((1, D_OUT)),
          _full((2 * D_OUT, D_OUT)), _full((1, 2 * D_OUT)),
          _full((1, 2 * D_OUT)),
          _full((2 * D_OUT, D_IN)), _full((1, 2 * D_OUT)),
          _full((1, 2 * D_OUT)),
      ],
      out_specs=pl.BlockSpec((_TF, 2 * D_OUT), grid_nk),
      out_shape=jax.ShapeDtypeStruct((N, 2 * D_OUT), _F32),
  )(z2, x, az2[None, :], cz2[None, :],
    W_mlp2, a3[None, :], c3[None, :],
    W_sc, asc[None, :], csc[None, :])

  return out.T[None, :, :, None]


# speedup vs baseline: 1.1040x; 1.1040x over previous
import functools

import jax
import jax.numpy as jnp
import numpy as np
from jax import lax
from jax.experimental import pallas as pl
from jax.experimental.pallas import tpu as pltpu
from jax.experimental.pallas import tpu_sc as plsc

N = 10000
K = 32
NK = N * K
D_IN = 128
D_OUT = 128
H = 64
EPS = 1e-5

_NC = 2
_NS = 16
_NW = _NC * _NS
_CH = 128
_GRP = 8
_CPW = 80
_NKP = _NW * _CPW * _CH

_T = 200
_R = _T * K
_GRID = N // _T
_TE = 2000
_TF = 400

_F32 = jnp.float32
_BF16 = jnp.bfloat16
_HI = lax.Precision.HIGHEST


def _gather_rows(table, idx3, d):
  mesh = plsc.VectorSubcoreMesh(core_axis_name="c", subcore_axis_name="s")

  @functools.partial(
      pl.kernel,
      out_type=jax.ShapeDtypeStruct((_NKP, d), _F32),
      mesh=mesh,
      scratch_types=[
          pltpu.VMEM((_CPW, _CH), jnp.int32),
          pltpu.VMEM((_GRP * _CH, d), _F32),
          pltpu.SemaphoreType.DMA,
      ],
      compiler_params=pltpu.CompilerParams(use_tc_tiling_on_sc=False),
  )
  def k(table_hbm, idx_hbm, out_hbm, idx_v, rows_v, sem):
    cid = lax.axis_index("c")
    sid = lax.axis_index("s")
    wid = sid * _NC + cid
    base = wid * _CPW * _CH
    pltpu.sync_copy(idx_hbm.at[wid], idx_v)

    def body(g, carry):
      cps = []
      for b in range(_GRP):
        cps.append(pltpu.async_copy(
            table_hbm.at[idx_v.at[g * _GRP + b]],
            rows_v.at[pl.ds(b * _CH, _CH)], sem))
      for cp in cps:
        cp.wait()
      pltpu.sync_copy(rows_v,
                      out_hbm.at[pl.ds(base + g * (_GRP * _CH), _GRP * _CH)])
      return carry

    lax.fori_loop(0, _CPW // _GRP, body, 0)

  return k(table, idx3)


_A_NP = np.zeros((16, 16), np.float32)
_B_NP = np.zeros((16, 16), np.float32)
for _i in range(3):
  _A_NP[_i, 1 + _i] = 1.0
  _A_NP[_i, 4 + _i] = 1.0
  _B_NP[_i, 1 + _i] = -1.0
  _B_NP[_i, 7 + _i] = 1.0


def _dotT(a, b):
  return lax.dot_general(a, b, (((1,), (1,)), ((), ())),
                         preferred_element_type=_F32, precision=_HI)


def _rel_dis(xt, nb):
  xtr = jnp.broadcast_to(xt[:, None, :], (_T, K, 16)).reshape(_R, 16)
  rel = xtr - nb
  dis = jnp.sqrt(jnp.sum(rel * rel, axis=1, keepdims=True))
  return rel, dis


def _kernel_ab(x_ref, xyz_ref, nxyz_ref, w1_ref, am_ref, bm_ref,
               y0_ref, s0_ref, sx_ref, m1_ref, s1_ref):
  i = pl.program_id(0)
  x = x_ref[...]
  y0_ref[...] = _dotT(x, w1_ref[...])

  @pl.when(i == 0)
  def _():
    s0_ref[...] = jnp.zeros_like(s0_ref)
    sx_ref[...] = jnp.zeros_like(sx_ref)
    m1_ref[...] = jnp.zeros_like(m1_ref)
    s1_ref[...] = jnp.zeros_like(s1_ref)

  s0_ref[...] += lax.dot_general(x, x, (((0,), (0,)), ((), ())),
                                 preferred_element_type=_F32, precision=_HI)
  sx_ref[...] += jnp.sum(x, axis=0, keepdims=True)

  nb = nxyz_ref[...]
  xt = xyz_ref[...]
  xtr = jnp.broadcast_to(xt[:, None, :], (_T, K, 16)).reshape(_R, 16)
  rel = xtr - nb
  dis = jnp.sqrt(jnp.sum(rel * rel, axis=1, keepdims=True))
  li = lax.broadcasted_iota(jnp.int32, (_R, 16), 1)
  f10 = (lax.dot_general(xtr, am_ref[...], (((1,), (0,)), ((), ())),
                         preferred_element_type=_F32, precision=_HI)
         + lax.dot_general(nb, bm_ref[...], (((1,), (0,)), ((), ())),
                           preferred_element_type=_F32, precision=_HI)
         + jnp.where(li == 0, dis, 0.0))
  m1_ref[...] += lax.dot_general(f10, f10, (((0,), (0,)), ((), ())),
                                 preferred_element_type=_F32, precision=_HI)
  s1_ref[...] += jnp.sum(f10, axis=0, keepdims=True)


def _fxyz(xyz_ref, nxyz_ref, wxt_ref, wnb_ref, w0_ref,
          a1_ref, c1_ref):
  nb = nxyz_ref[...]
  xt = xyz_ref[...]
  rel, dis = _rel_dis(xt, nb)
  px = lax.dot_general(xt, wxt_ref[...], (((1,), (0,)), ((), ())),
                       preferred_element_type=_F32, precision=_HI)
  pxb = jnp.broadcast_to(px[:, None, :], (_T, K, H)).reshape(_R, H)
  pn = lax.dot_general(nb, wnb_ref[...], (((1,), (0,)), ((), ())),
                       preferred_element_type=_F32, precision=_HI)
  y1 = pxb + pn + dis * w0_ref[...]
  return jax.nn.relu(y1 * a1_ref[...] + c1_ref[...])


def _att_pool_tile(fn, fx, wfca_ref, wfcb_ref, wml_ref, wmr_ref):
  att = _dotT(fn, wfca_ref[...]) + _dotT(fx, wfcb_ref[...])
  a3 = att.reshape(_T, K, D_OUT)
  m = jnp.max(a3, axis=1, keepdims=True)
  e = jnp.exp(a3 - m)
  sc = e / jnp.sum(e, axis=1, keepdims=True)
  fn3 = fn.reshape(_T, K, H)
  fx3 = fx.reshape(_T, K, H)
  agg_l = jnp.sum(fn3 * sc[:, :, :H], axis=1)
  agg_r = jnp.sum(fx3 * sc[:, :, H:], axis=1)
  return _dotT(agg_l, wml_ref[...]) + _dotT(agg_r, wmr_ref[...])


def _kernel_att1(xyz_ref, nxyz_ref, nb1_ref, wxt_ref, wnb_ref,
                 w0_ref, a1_ref, c1_ref, a0_ref, c0_ref,
                 wfca_ref, wfcb_ref, wml_ref, wmr_ref,
                 z1_ref, sz_ref, qz_ref, m2_ref, s2_ref):
  i = pl.program_id(0)
  fx = _fxyz(xyz_ref, nxyz_ref, wxt_ref, wnb_ref, w0_ref,
             a1_ref, c1_ref)
  fn = jax.nn.relu(nb1_ref[...] * a0_ref[...] + c0_ref[...])
  z1 = _att_pool_tile(fn, fx, wfca_ref, wfcb_ref, wml_ref, wmr_ref)
  z1_ref[...] = z1

  @pl.when(i == 0)
  def _():
    sz_ref[...] = jnp.zeros_like(sz_ref)
    qz_ref[...] = jnp.zeros_like(qz_ref)
    m2_ref[...] = jnp.zeros_like(m2_ref)
    s2_ref[...] = jnp.zeros_like(s2_ref)

  sz_ref[...] += jnp.sum(z1, axis=0, keepdims=True)
  qz_ref[...] += jnp.sum(z1 * z1, axis=0, keepdims=True)
  m2_ref[...] += lax.dot_general(fx, fx, (((0,), (0,)), ((), ())),
                                 preferred_element_type=_F32, precision=_HI)
  s2_ref[...] += jnp.sum(fx, axis=0, keepdims=True)


def _kernel_att2(xyz_ref, nxyz_ref, nb2_ref, wxt_ref, wnb_ref,
                 w0_ref, a1_ref, c1_ref, w2_ref, a2_ref, c2_ref,
                 az_ref, cz_ref, wfca_ref, wfcb_ref, wml_ref, wmr_ref,
                 z2_ref, sz_ref, qz_ref):
  i = pl.program_id(0)
  fx1 = _fxyz(xyz_ref, nxyz_ref, wxt_ref, wnb_ref, w0_ref,
              a1_ref, c1_ref)
  fx2 = jax.nn.relu(_dotT(fx1, w2_ref[...]) * a2_ref[...] + c2_ref[...])
  fn2 = jax.nn.relu(nb2_ref[...] * az_ref[...] + cz_ref[...])
  z2 = _att_pool_tile(fn2, fx2, wfca_ref, wfcb_ref, wml_ref, wmr_ref)
  z2_ref[...] = z2

  @pl.when(i == 0)
  def _():
    sz_ref[...] = jnp.zeros_like(sz_ref)
    qz_ref[...] = jnp.zeros_like(qz_ref)

  sz_ref[...] += jnp.sum(z2, axis=0, keepdims=True)
  qz_ref[...] += jnp.sum(z2 * z2, axis=0, keepdims=True)


def _kernel_stats3(z2_ref, az2_ref, cz2_ref, m3_ref, s3_ref):
  i = pl.program_id(0)
  fa = jax.nn.relu(z2_ref[...] * az2_ref[...] + cz2_ref[...])

  @pl.when(i == 0)
  def _():
    m3_ref[...] = jnp.zeros_like(m3_ref)
    s3_ref[...] = jnp.zeros_like(s3_ref)

  m3_ref[...] += lax.dot_general(fa, fa, (((0,), (0,)), ((), ())),
                                 preferred_element_type=_F32, precision=_HI)
  s3_ref[...] += jnp.sum(fa, axis=0, keepdims=True)


def _kernel_final(z2_ref, x_ref, az2_ref, cz2_ref, w3_ref, a3_ref, b3_ref,
                  wsc_ref, asc_ref, bsc_ref, out_ref):
  fa = jax.nn.relu(z2_ref[...] * az2_ref[...] + cz2_ref[...])
  y3 = _dotT(fa, w3_ref[...]) * a3_ref[...] + b3_ref[...]
  ysc = _dotT(x_ref[...], wsc_ref[...]) * asc_ref[...] + bsc_ref[...]
  o = y3 + ysc
  out_ref[...] = jnp.where(o >= 0, o, 0.2 * o)


def _full(shape):
  return pl.BlockSpec(shape, lambda i: (0,) * len(shape))


def _fold_moment(w, g, b, s2m, s1m, n):
  mu = s1m / n
  mean_y = jnp.matmul(w, mu, precision=_HI)
  ey2 = jnp.sum(jnp.matmul(w, s2m / n, precision=_HI) * w, axis=1)
  var = jnp.maximum(ey2 - mean_y * mean_y, 0.0)
  a = g * lax.rsqrt(var + EPS)
  return a, b - mean_y * a


def _fold_direct(g, b, ssum, sq, n):
  mean = ssum / n
  var = jnp.maximum(sq / n - mean * mean, 0.0)
  a = g * lax.rsqrt(var + EPS)
  return a, b - mean * a


def kernel(feature, xyz, neigh_idx, W_mlp1, g_mlp1, b_mlp1, W_bb1, g_bb1,
           b_bb1, W_att1_fc, W_att1_mlp, g_att1, b_att1, W_bb2, g_bb2, b_bb2,
           W_att2_fc, W_att2_mlp, g_att2, b_att2, W_mlp2, g_mlp2, b_mlp2,
           W_sc, g_sc, b_sc):
  x = feature[0, :, :, 0].T
  xyz16 = jnp.pad(xyz[0], ((0, 0), (0, 13)))
  idxf = neigh_idx[0].reshape(-1).astype(jnp.int32)
  idx3 = jnp.pad(idxf, (0, _NKP - NK)).reshape(_NW, _CPW, _CH)

  nxyz = _gather_rows(xyz16, idx3, 16)

  grid_nk = lambda i: (i, 0)
  y0, s0, sx, m1, s1 = pl.pallas_call(
      _kernel_ab,
      grid=(_GRID,),
      in_specs=[
          pl.BlockSpec((_T, D_IN), grid_nk),
          pl.BlockSpec((_T, 16), grid_nk),
          pl.BlockSpec((_R, 16), grid_nk),
          _full((H, D_IN)),
          _full((16, 16)), _full((16, 16)),
      ],
      out_specs=[
          pl.BlockSpec((_T, H), grid_nk),
          _full((D_IN, D_IN)),
          _full((1, D_IN)),
          _full((16, 16)),
          _full((1, 16)),
      ],
      out_shape=[
          jax.ShapeDtypeStruct((N, H), _F32),
          jax.ShapeDtypeStruct((D_IN, D_IN), _F32),
          jax.ShapeDtypeStruct((1, D_IN), _F32),
          jax.ShapeDtypeStruct((16, 16), _F32),
          jax.ShapeDtypeStruct((1, 16), _F32),
      ],
  )(x, xyz16, nxyz, W_mlp1,
    jnp.asarray(_A_NP), jnp.asarray(_B_NP))

  a0, c0 = _fold_moment(W_mlp1, g_mlp1, b_mlp1, s0, sx[0], N)
  w1p = jnp.pad(W_bb1, ((0, 0), (0, 6)))
  a1, c1 = _fold_moment(w1p, g_bb1, b_bb1, m1, s1[0], NK)
  w1b = W_bb1
  zpad = jnp.zeros((13, H), _F32)
  wrel = jnp.concatenate([w1b[:, 1:4].T, zpad], axis=0)
  wxt = jnp.concatenate([w1b[:, 4:7].T, zpad], axis=0) + wrel
  wnb = jnp.concatenate([w1b[:, 7:10].T, zpad], axis=0) - wrel
  w0row = w1b[:, 0][None, :]

  nb1 = _gather_rows(y0, idx3, H)

  z1, sz1, qz1, m2, s2 = pl.pallas_call(
      _kernel_att1,
      grid=(_GRID,),
      in_specs=[
          pl.BlockSpec((_T, 16), grid_nk),
          pl.BlockSpec((_R, 16), grid_nk),
          pl.BlockSpec((_R, H), grid_nk),
          _full((16, H)), _full((16, H)), _full((1, H)),
          _full((1, H)), _full((1, H)),
          _full((1, H)), _full((1, H)),
          _full((D_OUT, H)), _full((D_OUT, H)),
          _full((H, H)), _full((H, H)),
      ],
      out_specs=[
          pl.BlockSpec((_T, H), grid_nk),
          _full((1, H)), _full((1, H)), _full((H, H)), _full((1, H)),
      ],
      out_shape=[
          jax.ShapeDtypeStruct((N, H), _F32),
          jax.ShapeDtypeStruct((1, H), _F32),
          jax.ShapeDtypeStruct((1, H), _F32),
          jax.ShapeDtypeStruct((H, H), _F32),
          jax.ShapeDtypeStruct((1, H), _F32),
      ],
  )(xyz16, nxyz, nb1, wxt, wnb, w0row,
    a1[None, :], c1[None, :], a0[None, :], c0[None, :],
    W_att1_fc[:, :H], W_att1_fc[:, H:],
    W_att1_mlp[:, :H], W_att1_mlp[:, H:])

  az1, cz1 = _fold_direct(g_att1, b_att1, sz1[0], qz1[0], N)
  a2, c2 = _fold_moment(W_bb2, g_bb2, b_bb2, m2, s2[0], NK)

  nb2 = _gather_rows(z1, idx3, H)

  z2, sz2, qz2 = pl.pallas_call(
      _kernel_att2,
      grid=(_GRID,),
      in_specs=[
          pl.BlockSpec((_T, 16), grid_nk),
          pl.BlockSpec((_R, 16), grid_nk),
          pl.BlockSpec((_R, H), grid_nk),
          _full((16, H)), _full((16, H)), _full((1, H)),
          _full((1, H)), _full((1, H)),
          _full((H, H)), _full((1, H)), _full((1, H)),
          _full((1, H)), _full((1, H)),
          _full((D_OUT, H)), _full((D_OUT, H)),
          _full((D_OUT, H)), _full((D_OUT, H)),
      ],
      out_specs=[
          pl.BlockSpec((_T, D_OUT), grid_nk),
          _full((1, D_OUT)), _full((1, D_OUT)),
      ],
      out_shape=[
          jax.ShapeDtypeStruct((N, D_OUT), _F32),
          jax.ShapeDtypeStruct((1, D_OUT), _F32),
          jax.ShapeDtypeStruct((1, D_OUT), _F32),
      ],
  )(xyz16, nxyz, nb2, wxt, wnb, w0row,
    a1[None, :], c1[None, :],
    W_bb2, a2[None, :], c2[None, :],
    az1[None, :], cz1[None, :],
    W_att2_fc[:, :H], W_att2_fc[:, H:],
    W_att2_mlp[:, :H], W_att2_mlp[:, H:])

  az2, cz2 = _fold_direct(g_att2, b_att2, sz2[0], qz2[0], N)

  m3, s3 = pl.pallas_call(
      _kernel_stats3,
      grid=(N // _TE,),
      in_specs=[
          pl.BlockSpec((_TE, D_OUT), grid_nk),
          _full((1, D_OUT)), _full((1, D_OUT)),
      ],
      out_specs=[_full((D_OUT, D_OUT)), _full((1, D_OUT))],
      out_shape=[
          jax.ShapeDtypeStruct((D_OUT, D_OUT), _F32),
          jax.ShapeDtypeStruct((1, D_OUT), _F32),
      ],
  )(z2, az2[None, :], cz2[None, :])

  a3, c3 = _fold_moment(W_mlp2, g_mlp2, b_mlp2, m3, s3[0], N)
  asc, csc = _fold_moment(W_sc, g_sc, b_sc, s0, sx[0], N)

  out = pl.pallas_call(
      _kernel_final,
      grid=(N // _TF,),
      in_specs=[
          pl.BlockSpec((_TF, D_OUT), grid_nk),
          pl.BlockSpec((_TF, D_IN), grid_nk),
          _full((1, D_OUT)), _full((1, D_OUT)),
          _full((2 * D_OUT, D_OUT)), _full((1, 2 * D_OUT)),
          _full((1, 2 * D_OUT)),
          _full((2 * D_OUT, D_IN)), _full((1, 2 * D_OUT)),
          _full((1, 2 * D_OUT)),
      ],
      out_specs=pl.BlockSpec((_TF, 2 * D_OUT), grid_nk),
      out_shape=jax.ShapeDtypeStruct((N, 2 * D_OUT), _F32),
  )(z2, x, az2[None, :], cz2[None, :],
    W_mlp2, a3[None, :], c3[None, :],
    W_sc, asc[None, :], csc[None, :])

  return out.T[None, :, :, None]


# speedup vs baseline: 1.9637x; 1.7787x over previous
import functools

import jax
import jax.numpy as jnp
import numpy as np
from jax import lax
from jax.experimental import pallas as pl
from jax.experimental.pallas import tpu as pltpu
from jax.experimental.pallas import tpu_sc as plsc

N = 10000
K = 32
NK = N * K
D_IN = 128
D_OUT = 128
H = 64
EPS = 1e-5

_NC = 2
_NS = 16
_NW = _NC * _NS
_CH = 128
_GRP = 8
_CPW = 80
_NKP = _NW * _CPW * _CH

_T = 200
_R = _T * K
_GRID = N // _T
_TE = 2000
_TF = 400

_F32 = jnp.float32
_BF16 = jnp.bfloat16
_HI = lax.Precision.HIGHEST


def _gather_rows(table, idx3, d):
  mesh = plsc.VectorSubcoreMesh(core_axis_name="c", subcore_axis_name="s")

  @functools.partial(
      pl.kernel,
      out_type=jax.ShapeDtypeStruct((_NKP, d), _F32),
      mesh=mesh,
      scratch_types=[
          pltpu.VMEM((_CPW, _CH), jnp.int32),
          pltpu.VMEM((_GRP * _CH, d), _F32),
          pltpu.SemaphoreType.DMA,
      ],
      compiler_params=pltpu.CompilerParams(use_tc_tiling_on_sc=False),
  )
  def k(table_hbm, idx_hbm, out_hbm, idx_v, rows_v, sem):
    cid = lax.axis_index("c")
    sid = lax.axis_index("s")
    wid = sid * _NC + cid
    base = wid * _CPW * _CH
    pltpu.sync_copy(idx_hbm.at[wid], idx_v)

    def body(g, carry):
      cps = []
      for b in range(_GRP):
        cps.append(pltpu.async_copy(
            table_hbm.at[idx_v.at[g * _GRP + b]],
            rows_v.at[pl.ds(b * _CH, _CH)], sem))
      for cp in cps:
        cp.wait()
      pltpu.sync_copy(rows_v,
                      out_hbm.at[pl.ds(base + g * (_GRP * _CH), _GRP * _CH)])
      return carry

    lax.fori_loop(0, _CPW // _GRP, body, 0)

  return k(table, idx3)


_A_NP = np.zeros((16, 16), np.float32)
_B_NP = np.zeros((16, 16), np.float32)
for _i in range(3):
  _A_NP[_i, 1 + _i] = 1.0
  _A_NP[_i, 4 + _i] = 1.0
  _B_NP[_i, 1 + _i] = -1.0
  _B_NP[_i, 7 + _i] = 1.0


def _dotT(a, b):
  return lax.dot_general(a.astype(_BF16), b.astype(_BF16),
                         (((1,), (1,)), ((), ())),
                         preferred_element_type=_F32)


def _dotN(a, b):
  return lax.dot_general(a.astype(_BF16), b.astype(_BF16),
                         (((1,), (0,)), ((), ())),
                         preferred_element_type=_F32)


def _rel_dis(xt, nb):
  xtr = jnp.broadcast_to(xt[:, None, :], (_T, K, 16)).reshape(_R, 16)
  rel = xtr - nb
  dis = jnp.sqrt(jnp.sum(rel * rel, axis=1, keepdims=True))
  return rel, dis


def _kernel_ab(x_ref, xyz_ref, nxyz_ref, w1_ref, am_ref, bm_ref,
               y0_ref, s0_ref, sx_ref, m1_ref, s1_ref):
  i = pl.program_id(0)
  x = x_ref[...]
  y0_ref[...] = _dotT(x, w1_ref[...])

  @pl.when(i == 0)
  def _():
    s0_ref[...] = jnp.zeros_like(s0_ref)
    sx_ref[...] = jnp.zeros_like(sx_ref)
    m1_ref[...] = jnp.zeros_like(m1_ref)
    s1_ref[...] = jnp.zeros_like(s1_ref)

  s0_ref[...] += lax.dot_general(x, x, (((0,), (0,)), ((), ())),
                                 preferred_element_type=_F32, precision=_HI)
  sx_ref[...] += jnp.sum(x, axis=0, keepdims=True)

  nb = nxyz_ref[...]
  xt = xyz_ref[...]
  xtr = jnp.broadcast_to(xt[:, None, :], (_T, K, 16)).reshape(_R, 16)
  rel = xtr - nb
  dis = jnp.sqrt(jnp.sum(rel * rel, axis=1, keepdims=True))
  li = lax.broadcasted_iota(jnp.int32, (_R, 16), 1)
  f10 = (lax.dot_general(xtr, am_ref[...], (((1,), (0,)), ((), ())),
                         preferred_element_type=_F32, precision=_HI)
         + lax.dot_general(nb, bm_ref[...], (((1,), (0,)), ((), ())),
                           preferred_element_type=_F32, precision=_HI)
         + jnp.where(li == 0, dis, 0.0))
  m1_ref[...] += lax.dot_general(f10, f10, (((0,), (0,)), ((), ())),
                                 preferred_element_type=_F32, precision=_HI)
  s1_ref[...] += jnp.sum(f10, axis=0, keepdims=True)


def _fxyz(xyz_ref, nxyz_ref, wxt_ref, wnb_ref, w0_ref, a1_ref, c1_ref):
  nb = nxyz_ref[...]
  xt = xyz_ref[...]
  rel, dis = _rel_dis(xt, nb)
  px = _dotN(xt, wxt_ref[...])
  pxb = jnp.broadcast_to(px[:, None, :], (_T, K, D_OUT)).reshape(_R, D_OUT)
  pn = _dotN(nb, wnb_ref[...])
  y1 = pxb + pn + dis * w0_ref[...]
  return jax.nn.relu(y1 * a1_ref[...] + c1_ref[...])


def _att_pool_tile(fcat, wfc_ref, wm_ref):
  att = _dotT(fcat, wfc_ref[...])
  a3 = att.reshape(_T, K, D_OUT)
  m = jnp.max(a3, axis=1, keepdims=True)
  e = jnp.exp(a3 - m)
  sc = e / jnp.sum(e, axis=1, keepdims=True)
  agg = jnp.sum(fcat.reshape(_T, K, D_OUT) * sc, axis=1)
  return _dotT(agg, wm_ref[...])


def _kernel_att1(xyz_ref, nxyz_ref, nb1_ref, wxt_ref, wnb_ref,
                 w0_ref, a1_ref, c1_ref, a0_ref, c0_ref,
                 wfc_ref, wm_ref,
                 z1_ref, sz_ref, qz_ref, mf_ref, sf_ref):
  i = pl.program_id(0)
  fx = _fxyz(xyz_ref, nxyz_ref, wxt_ref, wnb_ref, w0_ref, a1_ref, c1_ref)
  fn = jax.nn.relu(nb1_ref[...] * a0_ref[...] + c0_ref[...])
  fcat = jnp.pad(fn, ((0, 0), (0, H))) + fx
  z1 = _att_pool_tile(fcat, wfc_ref, wm_ref)
  z1_ref[...] = z1

  @pl.when(i == 0)
  def _():
    sz_ref[...] = jnp.zeros_like(sz_ref)
    qz_ref[...] = jnp.zeros_like(qz_ref)
    mf_ref[...] = jnp.zeros_like(mf_ref)
    sf_ref[...] = jnp.zeros_like(sf_ref)

  sz_ref[...] += jnp.sum(z1, axis=0, keepdims=True)
  qz_ref[...] += jnp.sum(z1 * z1, axis=0, keepdims=True)
  mf_ref[...] += lax.dot_general(fcat, fcat, (((0,), (0,)), ((), ())),
                                 preferred_element_type=_F32, precision=_HI)
  sf_ref[...] += jnp.sum(fcat, axis=0, keepdims=True)


def _kernel_att2(xyz_ref, nxyz_ref, nb2_ref, wxt_ref, wnb_ref,
                 w0_ref, a1_ref, c1_ref, w2_ref, a2_ref, c2_ref,
                 az_ref, cz_ref, wfc_ref, wm_ref,
                 z2_ref, sz_ref, qz_ref):
  i = pl.program_id(0)
  fx1 = _fxyz(xyz_ref, nxyz_ref, wxt_ref, wnb_ref, w0_ref, a1_ref, c1_ref)
  fx2 = jax.nn.relu(_dotT(fx1, w2_ref[...]) * a2_ref[...] + c2_ref[...])
  fn2 = jax.nn.relu(nb2_ref[...] * az_ref[...] + cz_ref[...])
  fcat = jnp.pad(fn2, ((0, 0), (0, H))) + fx2
  z2 = _att_pool_tile(fcat, wfc_ref, wm_ref)
  z2_ref[...] = z2

  @pl.when(i == 0)
  def _():
    sz_ref[...] = jnp.zeros_like(sz_ref)
    qz_ref[...] = jnp.zeros_like(qz_ref)

  sz_ref[...] += jnp.sum(z2, axis=0, keepdims=True)
  qz_ref[...] += jnp.sum(z2 * z2, axis=0, keepdims=True)


def _kernel_stats3(z2_ref, az2_ref, cz2_ref, m3_ref, s3_ref):
  i = pl.program_id(0)
  fa = jax.nn.relu(z2_ref[...] * az2_ref[...] + cz2_ref[...])

  @pl.when(i == 0)
  def _():
    m3_ref[...] = jnp.zeros_like(m3_ref)
    s3_ref[...] = jnp.zeros_like(s3_ref)

  m3_ref[...] += lax.dot_general(fa, fa, (((0,), (0,)), ((), ())),
                                 preferred_element_type=_F32, precision=_HI)
  s3_ref[...] += jnp.sum(fa, axis=0, keepdims=True)


def _kernel_final(z2_ref, x_ref, az2_ref, cz2_ref, w3_ref, a3_ref, b3_ref,
                  wsc_ref, asc_ref, bsc_ref, out_ref):
  fa = jax.nn.relu(z2_ref[...] * az2_ref[...] + cz2_ref[...])
  y3 = _dotT(fa, w3_ref[...]) * a3_ref[...] + b3_ref[...]
  ysc = _dotT(x_ref[...], wsc_ref[...]) * asc_ref[...] + bsc_ref[...]
  o = y3 + ysc
  out_ref[...] = jnp.where(o >= 0, o, 0.2 * o)


def _full(shape):
  return pl.BlockSpec(shape, lambda i: (0,) * len(shape))


def _fold_moment(w, g, b, s2m, s1m, n):
  mu = s1m / n
  mean_y = jnp.matmul(w, mu, precision=_HI)
  ey2 = jnp.sum(jnp.matmul(w, s2m / n, precision=_HI) * w, axis=1)
  var = jnp.maximum(ey2 - mean_y * mean_y, 0.0)
  a = g * lax.rsqrt(var + EPS)
  return a, b - mean_y * a


def _fold_direct(g, b, ssum, sq, n):
  mean = ssum / n
  var = jnp.maximum(sq / n - mean * mean, 0.0)
  a = g * lax.rsqrt(var + EPS)
  return a, b - mean * a


def kernel(feature, xyz, neigh_idx, W_mlp1, g_mlp1, b_mlp1, W_bb1, g_bb1,
           b_bb1, W_att1_fc, W_att1_mlp, g_att1, b_att1, W_bb2, g_bb2, b_bb2,
           W_att2_fc, W_att2_mlp, g_att2, b_att2, W_mlp2, g_mlp2, b_mlp2,
           W_sc, g_sc, b_sc):
  x = feature[0, :, :, 0].T
  xyz16 = jnp.pad(xyz[0], ((0, 0), (0, 13)))
  idxf = neigh_idx[0].reshape(-1).astype(jnp.int32)
  idx3 = jnp.pad(idxf, (0, _NKP - NK)).reshape(_NW, _CPW, _CH)

  nxyz = _gather_rows(xyz16, idx3, 16)

  grid_nk = lambda i: (i, 0)
  y0, s0, sx, m1, s1 = pl.pallas_call(
      _kernel_ab,
      grid=(_GRID,),
      in_specs=[
          pl.BlockSpec((_T, D_IN), grid_nk),
          pl.BlockSpec((_T, 16), grid_nk),
          pl.BlockSpec((_R, 16), grid_nk),
          _full((H, D_IN)),
          _full((16, 16)), _full((16, 16)),
      ],
      out_specs=[
          pl.BlockSpec((_T, H), grid_nk),
          _full((D_IN, D_IN)),
          _full((1, D_IN)),
          _full((16, 16)),
          _full((1, 16)),
      ],
      out_shape=[
          jax.ShapeDtypeStruct((N, H), _F32),
          jax.ShapeDtypeStruct((D_IN, D_IN), _F32),
          jax.ShapeDtypeStruct((1, D_IN), _F32),
          jax.ShapeDtypeStruct((16, 16), _F32),
          jax.ShapeDtypeStruct((1, 16), _F32),
      ],
  )(x, xyz16, nxyz, W_mlp1,
    jnp.asarray(_A_NP), jnp.asarray(_B_NP))

  a0, c0 = _fold_moment(W_mlp1, g_mlp1, b_mlp1, s0, sx[0], N)
  w1p = jnp.pad(W_bb1, ((0, 0), (0, 6)))
  a1, c1 = _fold_moment(w1p, g_bb1, b_bb1, m1, s1[0], NK)
  w1b = W_bb1
  zpad = jnp.zeros((13, H), _F32)
  wrel = jnp.concatenate([w1b[:, 1:4].T, zpad], axis=0)
  wxt = jnp.concatenate([w1b[:, 4:7].T, zpad], axis=0) + wrel
  wnb = jnp.concatenate([w1b[:, 7:10].T, zpad], axis=0) - wrel
  wxt128 = jnp.pad(wxt, ((0, 0), (H, 0)))
  wnb128 = jnp.pad(wnb, ((0, 0), (H, 0)))
  w0row128 = jnp.pad(w1b[:, 0][None, :], ((0, 0), (H, 0)))
  a1p = jnp.pad(a1[None, :], ((0, 0), (H, 0)))
  c1p = jnp.pad(c1[None, :], ((0, 0), (H, 0)))

  nb1 = _gather_rows(y0, idx3, H)

  z1, sz1, qz1, mf, sf = pl.pallas_call(
      _kernel_att1,
      grid=(_GRID,),
      in_specs=[
          pl.BlockSpec((_T, 16), grid_nk),
          pl.BlockSpec((_R, 16), grid_nk),
          pl.BlockSpec((_R, H), grid_nk),
          _full((16, D_OUT)), _full((16, D_OUT)), _full((1, D_OUT)),
          _full((1, D_OUT)), _full((1, D_OUT)),
          _full((1, H)), _full((1, H)),
          _full((D_OUT, D_OUT)), _full((H, D_OUT)),
      ],
      out_specs=[
          pl.BlockSpec((_T, H), grid_nk),
          _full((1, H)), _full((1, H)),
          _full((D_OUT, D_OUT)), _full((1, D_OUT)),
      ],
      out_shape=[
          jax.ShapeDtypeStruct((N, H), _F32),
          jax.ShapeDtypeStruct((1, H), _F32),
          jax.ShapeDtypeStruct((1, H), _F32),
          jax.ShapeDtypeStruct((D_OUT, D_OUT), _F32),
          jax.ShapeDtypeStruct((1, D_OUT), _F32),
      ],
  )(xyz16, nxyz, nb1, wxt128, wnb128, w0row128,
    a1p, c1p, a0[None, :], c0[None, :],
    W_att1_fc, W_att1_mlp)

  az1, cz1 = _fold_direct(g_att1, b_att1, sz1[0], qz1[0], N)
  a2, c2 = _fold_moment(W_bb2, g_bb2, b_bb2, mf[H:, H:], sf[0, H:], NK)
  w2ext = jnp.pad(W_bb2, ((H, 0), (H, 0)))
  a2p = jnp.pad(a2[None, :], ((0, 0), (H, 0)))
  c2p = jnp.pad(c2[None, :], ((0, 0), (H, 0)))

  nb2 = _gather_rows(z1, idx3, H)

  z2, sz2, qz2 = pl.pallas_call(
      _kernel_att2,
      grid=(_GRID,),
      in_specs=[
          pl.BlockSpec((_T, 16), grid_nk),
          pl.BlockSpec((_R, 16), grid_nk),
          pl.BlockSpec((_R, H), grid_nk),
          _full((16, D_OUT)), _full((16, D_OUT)), _full((1, D_OUT)),
          _full((1, D_OUT)), _full((1, D_OUT)),
          _full((D_OUT, D_OUT)), _full((1, D_OUT)), _full((1, D_OUT)),
          _full((1, H)), _full((1, H)),
          _full((D_OUT, D_OUT)), _full((D_OUT, D_OUT)),
      ],
      out_specs=[
          pl.BlockSpec((_T, D_OUT), grid_nk),
          _full((1, D_OUT)), _full((1, D_OUT)),
      ],
      out_shape=[
          jax.ShapeDtypeStruct((N, D_OUT), _F32),
          jax.ShapeDtypeStruct((1, D_OUT), _F32),
          jax.ShapeDtypeStruct((1, D_OUT), _F32),
      ],
  )(xyz16, nxyz, nb2, wxt128, wnb128, w0row128,
    a1p, c1p,
    w2ext, a2p, c2p,
    az1[None, :], cz1[None, :],
    W_att2_fc, W_att2_mlp)

  az2, cz2 = _fold_direct(g_att2, b_att2, sz2[0], qz2[0], N)

  m3, s3 = pl.pallas_call(
      _kernel_stats3,
      grid=(N // _TE,),
      in_specs=[
          pl.BlockSpec((_TE, D_OUT), grid_nk),
          _full((1, D_OUT)), _full((1, D_OUT)),
      ],
      out_specs=[_full((D_OUT, D_OUT)), _full((1, D_OUT))],
      out_shape=[
          jax.ShapeDtypeStruct((D_OUT, D_OUT), _F32),
          jax.ShapeDtypeStruct((1, D_OUT), _F32),
      ],
  )(z2, az2[None, :], cz2[None, :])

  a3, c3 = _fold_moment(W_mlp2, g_mlp2, b_mlp2, m3, s3[0], N)
  asc, csc = _fold_moment(W_sc, g_sc, b_sc, s0, sx[0], N)

  out = pl.pallas_call(
      _kernel_final,
      grid=(N // _TF,),
      in_specs=[
          pl.BlockSpec((_TF, D_OUT), grid_nk),
          pl.BlockSpec((_TF, D_IN), grid_nk),
          _full((1, D_OUT)), _full((1, D_OUT)),
          _full((2 * D_OUT, D_OUT)), _full((1, 2 * D_OUT)),
          _full((1, 2 * D_OUT)),
          _full((2 * D_OUT, D_IN)), _full((1, 2 * D_OUT)),
          _full((1, 2 * D_OUT)),
      ],
      out_specs=pl.BlockSpec((_TF, 2 * D_OUT), grid_nk),
      out_shape=jax.ShapeDtypeStruct((N, 2 * D_OUT), _F32),
  )(z2, x, az2[None, :], cz2[None, :],
    W_mlp2, a3[None, :], c3[None, :],
    W_sc, asc[None, :], csc[None, :])

  return out.T[None, :, :, None]


# speedup vs baseline: 2.2896x; 1.1660x over previous
import functools

import jax
import jax.numpy as jnp
import numpy as np
from jax import lax
from jax.experimental import pallas as pl
from jax.experimental.pallas import tpu as pltpu
from jax.experimental.pallas import tpu_sc as plsc

N = 10000
K = 32
NK = N * K
D_IN = 128
D_OUT = 128
H = 64
EPS = 1e-5

_NC = 2
_NS = 16
_NW = _NC * _NS
_CH = 128
_GRP = 8
_CPW = 80
_NKP = _NW * _CPW * _CH

_T = 200
_R = _T * K
_GRID = N // _T
_TE = 2000
_TF = 400

_F32 = jnp.float32
_BF16 = jnp.bfloat16
_HI = lax.Precision.HIGHEST


def _gather_rows(table, idx3, d, dtype=_F32):
  mesh = plsc.VectorSubcoreMesh(core_axis_name="c", subcore_axis_name="s")

  @functools.partial(
      pl.kernel,
      out_type=jax.ShapeDtypeStruct((_NKP, d), dtype),
      mesh=mesh,
      scratch_types=[
          pltpu.VMEM((_CPW, _CH), jnp.int32),
          pltpu.VMEM((_GRP * _CH, d), dtype),
          pltpu.SemaphoreType.DMA,
      ],
      compiler_params=pltpu.CompilerParams(use_tc_tiling_on_sc=False),
  )
  def k(table_hbm, idx_hbm, out_hbm, idx_v, rows_v, sem):
    cid = lax.axis_index("c")
    sid = lax.axis_index("s")
    wid = sid * _NC + cid
    base = wid * _CPW * _CH
    pltpu.sync_copy(idx_hbm.at[wid], idx_v)

    def body(g, carry):
      cps = []
      for b in range(_GRP):
        cps.append(pltpu.async_copy(
            table_hbm.at[idx_v.at[g * _GRP + b]],
            rows_v.at[pl.ds(b * _CH, _CH)], sem))
      for cp in cps:
        cp.wait()
      pltpu.sync_copy(rows_v,
                      out_hbm.at[pl.ds(base + g * (_GRP * _CH), _GRP * _CH)])
      return carry

    lax.fori_loop(0, _CPW // _GRP, body, 0)

  return k(table, idx3)


_A_NP = np.zeros((16, 16), np.float32)
_B_NP = np.zeros((16, 16), np.float32)
for _i in range(3):
  _A_NP[_i, 1 + _i] = 1.0
  _A_NP[_i, 4 + _i] = 1.0
  _B_NP[_i, 1 + _i] = -1.0
  _B_NP[_i, 7 + _i] = 1.0


def _dotT(a, b):
  return lax.dot_general(a.astype(_BF16), b.astype(_BF16),
                         (((1,), (1,)), ((), ())),
                         preferred_element_type=_F32)


def _dotN(a, b):
  return lax.dot_general(a.astype(_BF16), b.astype(_BF16),
                         (((1,), (0,)), ((), ())),
                         preferred_element_type=_F32)


def _rel_dis(xt, nb):
  xtr = jnp.broadcast_to(xt[:, None, :], (_T, K, 16)).reshape(_R, 16)
  rel = xtr - nb
  dis = jnp.sqrt(jnp.sum(rel * rel, axis=1, keepdims=True))
  return rel, dis


def _kernel_ab(x_ref, xyz_ref, nxyz_ref, w1_ref, am_ref, bm_ref,
               y0_ref, s0_ref, sx_ref, m1_ref, s1_ref):
  i = pl.program_id(0)
  x = x_ref[...]
  y0_ref[...] = _dotT(x, w1_ref[...]).astype(_BF16)

  @pl.when(i == 0)
  def _():
    s0_ref[...] = jnp.zeros_like(s0_ref)
    sx_ref[...] = jnp.zeros_like(sx_ref)
    m1_ref[...] = jnp.zeros_like(m1_ref)
    s1_ref[...] = jnp.zeros_like(s1_ref)

  s0_ref[...] += lax.dot_general(x, x, (((0,), (0,)), ((), ())),
                                 preferred_element_type=_F32, precision=_HI)
  sx_ref[...] += jnp.sum(x, axis=0, keepdims=True)

  nb = nxyz_ref[...]
  xt = xyz_ref[...]
  xtr = jnp.broadcast_to(xt[:, None, :], (_T, K, 16)).reshape(_R, 16)
  rel = xtr - nb
  dis = jnp.sqrt(jnp.sum(rel * rel, axis=1, keepdims=True))
  li = lax.broadcasted_iota(jnp.int32, (_R, 16), 1)
  f10 = (_dotN(xtr, am_ref[...]) + _dotN(nb, bm_ref[...])
         + jnp.where(li == 0, dis, 0.0))
  m1_ref[...] += lax.dot_general(f10, f10, (((0,), (0,)), ((), ())),
                                 preferred_element_type=_F32, precision=_HI)
  s1_ref[...] += jnp.sum(f10, axis=0, keepdims=True)


def _fxyz(xyz_ref, nxyz_ref, wxt_ref, wnb_ref, w0_ref, a1_ref, c1_ref):
  nb = nxyz_ref[...]
  xt = xyz_ref[...]
  rel, dis = _rel_dis(xt, nb)
  px = _dotN(xt, wxt_ref[...])
  pxb = jnp.broadcast_to(px[:, None, :], (_T, K, D_OUT)).reshape(_R, D_OUT)
  pn = _dotN(nb, wnb_ref[...])
  y1 = pxb + pn + dis * w0_ref[...]
  return jax.nn.relu(y1 * a1_ref[...] + c1_ref[...])


def _att_pool_tile(fcat, wfc_ref, wm_ref):
  att = _dotT(fcat, wfc_ref[...])
  a3 = att.reshape(_T, K, D_OUT)
  m = jnp.max(a3, axis=1, keepdims=True)
  e = jnp.exp(a3 - m)
  sc = e / jnp.sum(e, axis=1, keepdims=True)
  agg = jnp.sum(fcat.reshape(_T, K, D_OUT) * sc, axis=1)
  return _dotT(agg, wm_ref[...])


def _kernel_att1(xyz_ref, nxyz_ref, nb1_ref, wxt_ref, wnb_ref,
                 w0_ref, a1_ref, c1_ref, a0_ref, c0_ref,
                 wfc_ref, wm_ref,
                 z1_ref, sz_ref, qz_ref, mf_ref, sf_ref):
  i = pl.program_id(0)
  fx = _fxyz(xyz_ref, nxyz_ref, wxt_ref, wnb_ref, w0_ref, a1_ref, c1_ref)
  fn = jax.nn.relu(nb1_ref[...].astype(_F32) * a0_ref[...] + c0_ref[...])
  fcat = jnp.pad(fn, ((0, 0), (0, H))) + fx
  z1 = _att_pool_tile(fcat, wfc_ref, wm_ref)
  z1_ref[...] = z1

  @pl.when(i == 0)
  def _():
    sz_ref[...] = jnp.zeros_like(sz_ref)
    qz_ref[...] = jnp.zeros_like(qz_ref)
    mf_ref[...] = jnp.zeros_like(mf_ref)
    sf_ref[...] = jnp.zeros_like(sf_ref)

  sz_ref[...] += jnp.sum(z1, axis=0, keepdims=True)
  qz_ref[...] += jnp.sum(z1 * z1, axis=0, keepdims=True)
  mf_ref[...] += lax.dot_general(fcat, fcat, (((0,), (0,)), ((), ())),
                                 preferred_element_type=_F32, precision=_HI)
  sf_ref[...] += jnp.sum(fcat, axis=0, keepdims=True)


def _kernel_att2(xyz_ref, nxyz_ref, nb2_ref, wxt_ref, wnb_ref,
                 w0_ref, a1_ref, c1_ref, w2_ref, a2_ref, c2_ref,
                 az_ref, cz_ref, wfc_ref, wm_ref,
                 z2_ref, sz_ref, qz_ref):
  i = pl.program_id(0)
  fx1 = _fxyz(xyz_ref, nxyz_ref, wxt_ref, wnb_ref, w0_ref, a1_ref, c1_ref)
  fx2 = jax.nn.relu(_dotT(fx1, w2_ref[...]) * a2_ref[...] + c2_ref[...])
  fn2 = jax.nn.relu(nb2_ref[...].astype(_F32) * az_ref[...] + cz_ref[...])
  fcat = jnp.pad(fn2, ((0, 0), (0, H))) + fx2
  z2 = _att_pool_tile(fcat, wfc_ref, wm_ref)
  z2_ref[...] = z2

  @pl.when(i == 0)
  def _():
    sz_ref[...] = jnp.zeros_like(sz_ref)
    qz_ref[...] = jnp.zeros_like(qz_ref)

  sz_ref[...] += jnp.sum(z2, axis=0, keepdims=True)
  qz_ref[...] += jnp.sum(z2 * z2, axis=0, keepdims=True)


def _kernel_stats3(z2_ref, az2_ref, cz2_ref, m3_ref, s3_ref):
  i = pl.program_id(0)
  fa = jax.nn.relu(z2_ref[...] * az2_ref[...] + cz2_ref[...])

  @pl.when(i == 0)
  def _():
    m3_ref[...] = jnp.zeros_like(m3_ref)
    s3_ref[...] = jnp.zeros_like(s3_ref)

  m3_ref[...] += lax.dot_general(fa, fa, (((0,), (0,)), ((), ())),
                                 preferred_element_type=_F32, precision=_HI)
  s3_ref[...] += jnp.sum(fa, axis=0, keepdims=True)


def _kernel_final(z2_ref, x_ref, az2_ref, cz2_ref, w3_ref, a3_ref, b3_ref,
                  wsc_ref, asc_ref, bsc_ref, out_ref):
  fa = jax.nn.relu(z2_ref[...] * az2_ref[...] + cz2_ref[...])
  y3 = _dotT(fa, w3_ref[...]) * a3_ref[...] + b3_ref[...]
  ysc = _dotT(x_ref[...], wsc_ref[...]) * asc_ref[...] + bsc_ref[...]
  o = y3 + ysc
  out_ref[...] = jnp.where(o >= 0, o, 0.2 * o)


def _full(shape):
  return pl.BlockSpec(shape, lambda i: (0,) * len(shape))


def _fold_moment(w, g, b, s2m, s1m, n):
  mu = s1m / n
  mean_y = jnp.matmul(w, mu, precision=_HI)
  ey2 = jnp.sum(jnp.matmul(w, s2m / n, precision=_HI) * w, axis=1)
  var = jnp.maximum(ey2 - mean_y * mean_y, 0.0)
  a = g * lax.rsqrt(var + EPS)
  return a, b - mean_y * a


def _fold_direct(g, b, ssum, sq, n):
  mean = ssum / n
  var = jnp.maximum(sq / n - mean * mean, 0.0)
  a = g * lax.rsqrt(var + EPS)
  return a, b - mean * a


def kernel(feature, xyz, neigh_idx, W_mlp1, g_mlp1, b_mlp1, W_bb1, g_bb1,
           b_bb1, W_att1_fc, W_att1_mlp, g_att1, b_att1, W_bb2, g_bb2, b_bb2,
           W_att2_fc, W_att2_mlp, g_att2, b_att2, W_mlp2, g_mlp2, b_mlp2,
           W_sc, g_sc, b_sc):
  x = feature[0, :, :, 0].T
  xyz16 = jnp.pad(xyz[0], ((0, 0), (0, 13)))
  idxf = neigh_idx[0].reshape(-1).astype(jnp.int32)
  idx3 = jnp.pad(idxf, (0, _NKP - NK)).reshape(_NW, _CPW, _CH)

  nxyz = _gather_rows(xyz16, idx3, 16)

  grid_nk = lambda i: (i, 0)
  y0, s0, sx, m1, s1 = pl.pallas_call(
      _kernel_ab,
      grid=(_GRID,),
      in_specs=[
          pl.BlockSpec((_T, D_IN), grid_nk),
          pl.BlockSpec((_T, 16), grid_nk),
          pl.BlockSpec((_R, 16), grid_nk),
          _full((H, D_IN)),
          _full((16, 16)), _full((16, 16)),
      ],
      out_specs=[
          pl.BlockSpec((_T, H), grid_nk),
          _full((D_IN, D_IN)),
          _full((1, D_IN)),
          _full((16, 16)),
          _full((1, 16)),
      ],
      out_shape=[
          jax.ShapeDtypeStruct((N, H), _BF16),
          jax.ShapeDtypeStruct((D_IN, D_IN), _F32),
          jax.ShapeDtypeStruct((1, D_IN), _F32),
          jax.ShapeDtypeStruct((16, 16), _F32),
          jax.ShapeDtypeStruct((1, 16), _F32),
      ],
  )(x, xyz16, nxyz, W_mlp1,
    jnp.asarray(_A_NP), jnp.asarray(_B_NP))

  a0, c0 = _fold_moment(W_mlp1, g_mlp1, b_mlp1, s0, sx[0], N)
  w1p = jnp.pad(W_bb1, ((0, 0), (0, 6)))
  a1, c1 = _fold_moment(w1p, g_bb1, b_bb1, m1, s1[0], NK)
  w1b = W_bb1
  zpad = jnp.zeros((13, H), _F32)
  wrel = jnp.concatenate([w1b[:, 1:4].T, zpad], axis=0)
  wxt = jnp.concatenate([w1b[:, 4:7].T, zpad], axis=0) + wrel
  wnb = jnp.concatenate([w1b[:, 7:10].T, zpad], axis=0) - wrel
  wxt128 = jnp.pad(wxt, ((0, 0), (H, 0)))
  wnb128 = jnp.pad(wnb, ((0, 0), (H, 0)))
  w0row128 = jnp.pad(w1b[:, 0][None, :], ((0, 0), (H, 0)))
  a1p = jnp.pad(a1[None, :], ((0, 0), (H, 0)))
  c1p = jnp.pad(c1[None, :], ((0, 0), (H, 0)))

  nb1 = _gather_rows(y0, idx3, H, _BF16)

  z1, sz1, qz1, mf, sf = pl.pallas_call(
      _kernel_att1,
      grid=(_GRID,),
      in_specs=[
          pl.BlockSpec((_T, 16), grid_nk),
          pl.BlockSpec((_R, 16), grid_nk),
          pl.BlockSpec((_R, H), grid_nk),
          _full((16, D_OUT)), _full((16, D_OUT)), _full((1, D_OUT)),
          _full((1, D_OUT)), _full((1, D_OUT)),
          _full((1, H)), _full((1, H)),
          _full((D_OUT, D_OUT)), _full((H, D_OUT)),
      ],
      out_specs=[
          pl.BlockSpec((_T, H), grid_nk),
          _full((1, H)), _full((1, H)),
          _full((D_OUT, D_OUT)), _full((1, D_OUT)),
      ],
      out_shape=[
          jax.ShapeDtypeStruct((N, H), _F32),
          jax.ShapeDtypeStruct((1, H), _F32),
          jax.ShapeDtypeStruct((1, H), _F32),
          jax.ShapeDtypeStruct((D_OUT, D_OUT), _F32),
          jax.ShapeDtypeStruct((1, D_OUT), _F32),
      ],
  )(xyz16, nxyz, nb1, wxt128, wnb128, w0row128,
    a1p, c1p, a0[None, :], c0[None, :],
    W_att1_fc, W_att1_mlp)

  az1, cz1 = _fold_direct(g_att1, b_att1, sz1[0], qz1[0], N)
  a2, c2 = _fold_moment(W_bb2, g_bb2, b_bb2, mf[H:, H:], sf[0, H:], NK)
  w2ext = jnp.pad(W_bb2, ((H, 0), (H, 0)))
  a2p = jnp.pad(a2[None, :], ((0, 0), (H, 0)))
  c2p = jnp.pad(c2[None, :], ((0, 0), (H, 0)))

  nb2 = _gather_rows(z1.astype(_BF16), idx3, H, _BF16)

  z2, sz2, qz2 = pl.pallas_call(
      _kernel_att2,
      grid=(_GRID,),
      in_specs=[
          pl.BlockSpec((_T, 16), grid_nk),
          pl.BlockSpec((_R, 16), grid_nk),
          pl.BlockSpec((_R, H), grid_nk),
          _full((16, D_OUT)), _full((16, D_OUT)), _full((1, D_OUT)),
          _full((1, D_OUT)), _full((1, D_OUT)),
          _full((D_OUT, D_OUT)), _full((1, D_OUT)), _full((1, D_OUT)),
          _full((1, H)), _full((1, H)),
          _full((D_OUT, D_OUT)), _full((D_OUT, D_OUT)),
      ],
      out_specs=[
          pl.BlockSpec((_T, D_OUT), grid_nk),
          _full((1, D_OUT)), _full((1, D_OUT)),
      ],
      out_shape=[
          jax.ShapeDtypeStruct((N, D_OUT), _F32),
          jax.ShapeDtypeStruct((1, D_OUT), _F32),
          jax.ShapeDtypeStruct((1, D_OUT), _F32),
      ],
  )(xyz16, nxyz, nb2, wxt128, wnb128, w0row128,
    a1p, c1p,
    w2ext, a2p, c2p,
    az1[None, :], cz1[None, :],
    W_att2_fc, W_att2_mlp)

  az2, cz2 = _fold_direct(g_att2, b_att2, sz2[0], qz2[0], N)

  m3, s3 = pl.pallas_call(
      _kernel_stats3,
      grid=(N // _TE,),
      in_specs=[
          pl.BlockSpec((_TE, D_OUT), grid_nk),
          _full((1, D_OUT)), _full((1, D_OUT)),
      ],
      out_specs=[_full((D_OUT, D_OUT)), _full((1, D_OUT))],
      out_shape=[
          jax.ShapeDtypeStruct((D_OUT, D_OUT), _F32),
          jax.ShapeDtypeStruct((1, D_OUT), _F32),
      ],
  )(z2, az2[None, :], cz2[None, :])

  a3, c3 = _fold_moment(W_mlp2, g_mlp2, b_mlp2, m3, s3[0], N)
  asc, csc = _fold_moment(W_sc, g_sc, b_sc, s0, sx[0], N)

  out = pl.pallas_call(
      _kernel_final,
      grid=(N // _TF,),
      in_specs=[
          pl.BlockSpec((_TF, D_OUT), grid_nk),
          pl.BlockSpec((_TF, D_IN), grid_nk),
          _full((1, D_OUT)), _full((1, D_OUT)),
          _full((2 * D_OUT, D_OUT)), _full((1, 2 * D_OUT)),
          _full((1, 2 * D_OUT)),
          _full((2 * D_OUT, D_IN)), _full((1, 2 * D_OUT)),
          _full((1, 2 * D_OUT)),
      ],
      out_specs=pl.BlockSpec((_TF, 2 * D_OUT), grid_nk),
      out_shape=jax.ShapeDtypeStruct((N, 2 * D_OUT), _F32),
  )(z2, x, az2[None, :], cz2[None, :],
    W_mlp2, a3[None, :], c3[None, :],
    W_sc, asc[None, :], csc[None, :])

  return out.T[None, :, :, None]


# speedup vs baseline: 2.5830x; 1.1281x over previous
import functools

import jax
import jax.numpy as jnp
import numpy as np
from jax import lax
from jax.experimental import pallas as pl
from jax.experimental.pallas import tpu as pltpu
from jax.experimental.pallas import tpu_sc as plsc

N = 10000
K = 32
NK = N * K
D_IN = 128
D_OUT = 128
H = 64
EPS = 1e-5

_NC = 2
_NS = 16
_NW = _NC * _NS
_CH = 128
_GRP = 8
_CPW = 80
_NKP = _NW * _CPW * _CH

_T = 200
_R = _T * K
_GRID = N // _T
_TE = 2000
_TF = 400

_F32 = jnp.float32
_BF16 = jnp.bfloat16
_HI = lax.Precision.HIGHEST


def _gather_rows(table, idx3, d, dtype=_F32):
  mesh = plsc.VectorSubcoreMesh(core_axis_name="c", subcore_axis_name="s")

  @functools.partial(
      pl.kernel,
      out_type=jax.ShapeDtypeStruct((_NKP, d), dtype),
      mesh=mesh,
      scratch_types=[
          pltpu.VMEM((_CPW, _CH), jnp.int32),
          pltpu.VMEM((_GRP * _CH, d), dtype),
          pltpu.SemaphoreType.DMA,
      ],
      compiler_params=pltpu.CompilerParams(use_tc_tiling_on_sc=False),
  )
  def k(table_hbm, idx_hbm, out_hbm, idx_v, rows_v, sem):
    cid = lax.axis_index("c")
    sid = lax.axis_index("s")
    wid = sid * _NC + cid
    base = wid * _CPW * _CH
    pltpu.sync_copy(idx_hbm.at[wid], idx_v)

    def body(g, carry):
      cps = []
      for b in range(_GRP):
        cps.append(pltpu.async_copy(
            table_hbm.at[idx_v.at[g * _GRP + b]],
            rows_v.at[pl.ds(b * _CH, _CH)], sem))
      for cp in cps:
        cp.wait()
      pltpu.sync_copy(rows_v,
                      out_hbm.at[pl.ds(base + g * (_GRP * _CH), _GRP * _CH)])
      return carry

    lax.fori_loop(0, _CPW // _GRP, body, 0)

  return k(table, idx3)


_A_NP = np.zeros((16, 16), np.float32)
_B_NP = np.zeros((16, 16), np.float32)
for _i in range(3):
  _A_NP[_i, 1 + _i] = 1.0
  _A_NP[_i, 4 + _i] = 1.0
  _B_NP[_i, 1 + _i] = -1.0
  _B_NP[_i, 7 + _i] = 1.0


def _dotT(a, b):
  return lax.dot_general(a.astype(_BF16), b.astype(_BF16),
                         (((1,), (1,)), ((), ())),
                         preferred_element_type=_F32)


def _dotN(a, b):
  return lax.dot_general(a.astype(_BF16), b.astype(_BF16),
                         (((1,), (0,)), ((), ())),
                         preferred_element_type=_F32)


def _dotM(a, b):
  return lax.dot_general(a.astype(_BF16), b.astype(_BF16),
                         (((0,), (0,)), ((), ())),
                         preferred_element_type=_F32)


def _rel_dis(xt, nb):
  xtr = jnp.broadcast_to(xt[:, None, :], (_T, K, 16)).reshape(_R, 16)
  rel = xtr - nb
  dis = jnp.sqrt(jnp.sum(rel * rel, axis=1, keepdims=True))
  return rel, dis


def _kernel_ab(x_ref, xyz_ref, nxyz_ref, w1_ref, am_ref, bm_ref,
               y0_ref, s0_ref, sx_ref, m1_ref, s1_ref):
  i = pl.program_id(0)
  x = x_ref[...]
  y0_ref[...] = _dotT(x, w1_ref[...]).astype(_BF16)

  @pl.when(i == 0)
  def _():
    s0_ref[...] = jnp.zeros_like(s0_ref)
    sx_ref[...] = jnp.zeros_like(sx_ref)
    m1_ref[...] = jnp.zeros_like(m1_ref)
    s1_ref[...] = jnp.zeros_like(s1_ref)

  s0_ref[...] += _dotM(x, x)
  sx_ref[...] += jnp.sum(x, axis=0, keepdims=True)

  nb = nxyz_ref[...]
  xt = xyz_ref[...]
  xtr = jnp.broadcast_to(xt[:, None, :], (_T, K, 16)).reshape(_R, 16)
  rel = xtr - nb
  dis = jnp.sqrt(jnp.sum(rel * rel, axis=1, keepdims=True))
  li = lax.broadcasted_iota(jnp.int32, (_R, 16), 1)
  f10 = (_dotN(xtr, am_ref[...]) + _dotN(nb, bm_ref[...])
         + jnp.where(li == 0, dis, 0.0))
  m1_ref[...] += _dotM(f10, f10)
  s1_ref[...] += jnp.sum(f10, axis=0, keepdims=True)


def _fxyz(xyz_ref, nxyz_ref, wxt_ref, wnb_ref, w0_ref, a1_ref, c1_ref):
  nb = nxyz_ref[...]
  xt = xyz_ref[...]
  rel, dis = _rel_dis(xt, nb)
  px = _dotN(xt, wxt_ref[...])
  pxb = jnp.broadcast_to(px[:, None, :], (_T, K, D_OUT)).reshape(_R, D_OUT)
  pn = _dotN(nb, wnb_ref[...])
  y1 = pxb + pn + dis * w0_ref[...]
  return jax.nn.relu(y1 * a1_ref[...] + c1_ref[...])


def _att_pool_tile(fcat, wfc_ref, wm_ref):
  att = _dotT(fcat, wfc_ref[...])
  a3 = att.reshape(_T, K, D_OUT)
  m = jnp.max(a3, axis=1, keepdims=True)
  e = jnp.exp(a3 - m)
  sc = e / jnp.sum(e, axis=1, keepdims=True)
  agg = jnp.sum(fcat.reshape(_T, K, D_OUT) * sc, axis=1)
  return _dotT(agg, wm_ref[...])


def _kernel_att1(xyz_ref, nxyz_ref, nb1_ref, wxt_ref, wnb_ref,
                 w0_ref, a1_ref, c1_ref, a0_ref, c0_ref,
                 wfc_ref, wm_ref,
                 z1_ref, sz_ref, qz_ref, mf_ref, sf_ref):
  i = pl.program_id(0)
  fx = _fxyz(xyz_ref, nxyz_ref, wxt_ref, wnb_ref, w0_ref, a1_ref, c1_ref)
  fn = jax.nn.relu(nb1_ref[...].astype(_F32) * a0_ref[...] + c0_ref[...])
  fcat = jnp.pad(fn, ((0, 0), (0, H))) + fx
  z1 = _att_pool_tile(fcat, wfc_ref, wm_ref)
  z1_ref[...] = z1

  @pl.when(i == 0)
  def _():
    sz_ref[...] = jnp.zeros_like(sz_ref)
    qz_ref[...] = jnp.zeros_like(qz_ref)
    mf_ref[...] = jnp.zeros_like(mf_ref)
    sf_ref[...] = jnp.zeros_like(sf_ref)

  sz_ref[...] += jnp.sum(z1, axis=0, keepdims=True)
  qz_ref[...] += jnp.sum(z1 * z1, axis=0, keepdims=True)
  mf_ref[...] += _dotM(fcat, fcat)
  sf_ref[...] += jnp.sum(fcat, axis=0, keepdims=True)


def _kernel_att2(xyz_ref, nxyz_ref, nb2_ref, wxt_ref, wnb_ref,
                 w0_ref, a1_ref, c1_ref, w2_ref, a2_ref, c2_ref,
                 az_ref, cz_ref, wfc_ref, wm_ref,
                 z2_ref, sz_ref, qz_ref):
  i = pl.program_id(0)
  fx1 = _fxyz(xyz_ref, nxyz_ref, wxt_ref, wnb_ref, w0_ref, a1_ref, c1_ref)
  fx2 = jax.nn.relu(_dotT(fx1, w2_ref[...]) * a2_ref[...] + c2_ref[...])
  fn2 = jax.nn.relu(nb2_ref[...].astype(_F32) * az_ref[...] + cz_ref[...])
  fcat = jnp.pad(fn2, ((0, 0), (0, H))) + fx2
  z2 = _att_pool_tile(fcat, wfc_ref, wm_ref)
  z2_ref[...] = z2

  @pl.when(i == 0)
  def _():
    sz_ref[...] = jnp.zeros_like(sz_ref)
    qz_ref[...] = jnp.zeros_like(qz_ref)

  sz_ref[...] += jnp.sum(z2, axis=0, keepdims=True)
  qz_ref[...] += jnp.sum(z2 * z2, axis=0, keepdims=True)


def _kernel_stats3(z2_ref, az2_ref, cz2_ref, m3_ref, s3_ref):
  i = pl.program_id(0)
  fa = jax.nn.relu(z2_ref[...] * az2_ref[...] + cz2_ref[...])

  @pl.when(i == 0)
  def _():
    m3_ref[...] = jnp.zeros_like(m3_ref)
    s3_ref[...] = jnp.zeros_like(s3_ref)

  m3_ref[...] += _dotM(fa, fa)
  s3_ref[...] += jnp.sum(fa, axis=0, keepdims=True)


def _kernel_final(z2_ref, x_ref, az2_ref, cz2_ref, w3_ref, a3_ref, b3_ref,
                  wsc_ref, asc_ref, bsc_ref, out_ref):
  fa = jax.nn.relu(z2_ref[...] * az2_ref[...] + cz2_ref[...])
  y3 = _dotT(fa, w3_ref[...]) * a3_ref[...] + b3_ref[...]
  ysc = _dotT(x_ref[...], wsc_ref[...]) * asc_ref[...] + bsc_ref[...]
  o = y3 + ysc
  out_ref[...] = jnp.where(o >= 0, o, 0.2 * o)


def _full(shape):
  return pl.BlockSpec(shape, lambda i: (0,) * len(shape))


def _fold_moment(w, g, b, s2m, s1m, n):
  mu = s1m / n
  mean_y = jnp.matmul(w, mu, precision=_HI)
  ey2 = jnp.sum(jnp.matmul(w, s2m / n, precision=_HI) * w, axis=1)
  var = jnp.maximum(ey2 - mean_y * mean_y, 0.0)
  a = g * lax.rsqrt(var + EPS)
  return a, b - mean_y * a


def _fold_direct(g, b, ssum, sq, n):
  mean = ssum / n
  var = jnp.maximum(sq / n - mean * mean, 0.0)
  a = g * lax.rsqrt(var + EPS)
  return a, b - mean * a


def kernel(feature, xyz, neigh_idx, W_mlp1, g_mlp1, b_mlp1, W_bb1, g_bb1,
           b_bb1, W_att1_fc, W_att1_mlp, g_att1, b_att1, W_bb2, g_bb2, b_bb2,
           W_att2_fc, W_att2_mlp, g_att2, b_att2, W_mlp2, g_mlp2, b_mlp2,
           W_sc, g_sc, b_sc):
  x = feature[0, :, :, 0].T
  xyz16 = jnp.pad(xyz[0], ((0, 0), (0, 13)))
  idxf = neigh_idx[0].reshape(-1).astype(jnp.int32)
  idx3 = jnp.pad(idxf, (0, _NKP - NK)).reshape(_NW, _CPW, _CH)

  nxyz = _gather_rows(xyz16, idx3, 16)

  grid_nk = lambda i: (i, 0)
  y0, s0, sx, m1, s1 = pl.pallas_call(
      _kernel_ab,
      grid=(_GRID,),
      in_specs=[
          pl.BlockSpec((_T, D_IN), grid_nk),
          pl.BlockSpec((_T, 16), grid_nk),
          pl.BlockSpec((_R, 16), grid_nk),
          _full((H, D_IN)),
          _full((16, 16)), _full((16, 16)),
      ],
      out_specs=[
          pl.BlockSpec((_T, H), grid_nk),
          _full((D_IN, D_IN)),
          _full((1, D_IN)),
          _full((16, 16)),
          _full((1, 16)),
      ],
      out_shape=[
          jax.ShapeDtypeStruct((N, H), _BF16),
          jax.ShapeDtypeStruct((D_IN, D_IN), _F32),
          jax.ShapeDtypeStruct((1, D_IN), _F32),
          jax.ShapeDtypeStruct((16, 16), _F32),
          jax.ShapeDtypeStruct((1, 16), _F32),
      ],
  )(x, xyz16, nxyz, W_mlp1,
    jnp.asarray(_A_NP), jnp.asarray(_B_NP))

  a0, c0 = _fold_moment(W_mlp1, g_mlp1, b_mlp1, s0, sx[0], N)
  w1p = jnp.pad(W_bb1, ((0, 0), (0, 6)))
  a1, c1 = _fold_moment(w1p, g_bb1, b_bb1, m1, s1[0], NK)
  w1b = W_bb1
  zpad = jnp.zeros((13, H), _F32)
  wrel = jnp.concatenate([w1b[:, 1:4].T, zpad], axis=0)
  wxt = jnp.concatenate([w1b[:, 4:7].T, zpad], axis=0) + wrel
  wnb = jnp.concatenate([w1b[:, 7:10].T, zpad], axis=0) - wrel
  wxt128 = jnp.pad(wxt, ((0, 0), (H, 0)))
  wnb128 = jnp.pad(wnb, ((0, 0), (H, 0)))
  w0row128 = jnp.pad(w1b[:, 0][None, :], ((0, 0), (H, 0)))
  a1p = jnp.pad(a1[None, :], ((0, 0), (H, 0)))
  c1p = jnp.pad(c1[None, :], ((0, 0), (H, 0)))

  nb1 = _gather_rows(y0, idx3, H, _BF16)

  z1, sz1, qz1, mf, sf = pl.pallas_call(
      _kernel_att1,
      grid=(_GRID,),
      in_specs=[
          pl.BlockSpec((_T, 16), grid_nk),
          pl.BlockSpec((_R, 16), grid_nk),
          pl.BlockSpec((_R, H), grid_nk),
          _full((16, D_OUT)), _full((16, D_OUT)), _full((1, D_OUT)),
          _full((1, D_OUT)), _full((1, D_OUT)),
          _full((1, H)), _full((1, H)),
          _full((D_OUT, D_OUT)), _full((H, D_OUT)),
      ],
      out_specs=[
          pl.BlockSpec((_T, H), grid_nk),
          _full((1, H)), _full((1, H)),
          _full((D_OUT, D_OUT)), _full((1, D_OUT)),
      ],
      out_shape=[
          jax.ShapeDtypeStruct((N, H), _F32),
          jax.ShapeDtypeStruct((1, H), _F32),
          jax.ShapeDtypeStruct((1, H), _F32),
          jax.ShapeDtypeStruct((D_OUT, D_OUT), _F32),
          jax.ShapeDtypeStruct((1, D_OUT), _F32),
      ],
  )(xyz16, nxyz, nb1, wxt128, wnb128, w0row128,
    a1p, c1p, a0[None, :], c0[None, :],
    W_att1_fc, W_att1_mlp)

  az1, cz1 = _fold_direct(g_att1, b_att1, sz1[0], qz1[0], N)
  a2, c2 = _fold_moment(W_bb2, g_bb2, b_bb2, mf[H:, H:], sf[0, H:], NK)
  w2ext = jnp.pad(W_bb2, ((H, 0), (H, 0)))
  a2p = jnp.pad(a2[None, :], ((0, 0), (H, 0)))
  c2p = jnp.pad(c2[None, :], ((0, 0), (H, 0)))

  nb2 = _gather_rows(z1.astype(_BF16), idx3, H, _BF16)

  z2, sz2, qz2 = pl.pallas_call(
      _kernel_att2,
      grid=(_GRID,),
      in_specs=[
          pl.BlockSpec((_T, 16), grid_nk),
          pl.BlockSpec((_R, 16), grid_nk),
          pl.BlockSpec((_R, H), grid_nk),
          _full((16, D_OUT)), _full((16, D_OUT)), _full((1, D_OUT)),
          _full((1, D_OUT)), _full((1, D_OUT)),
          _full((D_OUT, D_OUT)), _full((1, D_OUT)), _full((1, D_OUT)),
          _full((1, H)), _full((1, H)),
          _full((D_OUT, D_OUT)), _full((D_OUT, D_OUT)),
      ],
      out_specs=[
          pl.BlockSpec((_T, D_OUT), grid_nk),
          _full((1, D_OUT)), _full((1, D_OUT)),
      ],
      out_shape=[
          jax.ShapeDtypeStruct((N, D_OUT), _F32),
          jax.ShapeDtypeStruct((1, D_OUT), _F32),
          jax.ShapeDtypeStruct((1, D_OUT), _F32),
      ],
  )(xyz16, nxyz, nb2, wxt128, wnb128, w0row128,
    a1p, c1p,
    w2ext, a2p, c2p,
    az1[None, :], cz1[None, :],
    W_att2_fc, W_att2_mlp)

  az2, cz2 = _fold_direct(g_att2, b_att2, sz2[0], qz2[0], N)

  m3, s3 = pl.pallas_call(
      _kernel_stats3,
      grid=(N // _TE,),
      in_specs=[
          pl.BlockSpec((_TE, D_OUT), grid_nk),
          _full((1, D_OUT)), _full((1, D_OUT)),
      ],
      out_specs=[_full((D_OUT, D_OUT)), _full((1, D_OUT))],
      out_shape=[
          jax.ShapeDtypeStruct((D_OUT, D_OUT), _F32),
          jax.ShapeDtypeStruct((1, D_OUT), _F32),
      ],
  )(z2, az2[None, :], cz2[None, :])

  a3, c3 = _fold_moment(W_mlp2, g_mlp2, b_mlp2, m3, s3[0], N)
  asc, csc = _fold_moment(W_sc, g_sc, b_sc, s0, sx[0], N)

  out = pl.pallas_call(
      _kernel_final,
      grid=(N // _TF,),
      in_specs=[
          pl.BlockSpec((_TF, D_OUT), grid_nk),
          pl.BlockSpec((_TF, D_IN), grid_nk),
          _full((1, D_OUT)), _full((1, D_OUT)),
          _full((2 * D_OUT, D_OUT)), _full((1, 2 * D_OUT)),
          _full((1, 2 * D_OUT)),
          _full((2 * D_OUT, D_IN)), _full((1, 2 * D_OUT)),
          _full((1, 2 * D_OUT)),
      ],
      out_specs=pl.BlockSpec((_TF, 2 * D_OUT), grid_nk),
      out_shape=jax.ShapeDtypeStruct((N, 2 * D_OUT), _F32),
  )(z2, x, az2[None, :], cz2[None, :],
    W_mlp2, a3[None, :], c3[None, :],
    W_sc, asc[None, :], csc[None, :])

  return out.T[None, :, :, None]
